# Initial kernel scaffold; baseline (speedup 1.0000x reference)
import jax
import jax.numpy as jnp
from jax.experimental import pallas as pl


def kernel(h_id, r_id, t_id, ent_transfer, rel_transfer):
    raise NotImplementedError("write your pallas kernel here")


# speedup vs baseline: 1.8109x; 1.8109x over previous
import functools

import jax
import jax.numpy as jnp
from jax import lax
from jax.experimental import pallas as pl
from jax.experimental.pallas import tpu as pltpu
from jax.experimental.pallas import tpu_sc as plsc

_B = 4096
_T = 50
_D = 64
_N = _B * _T
_NC = 2
_NS = 16
_NW = _NC * _NS
_PER_W = _N // _NW
_CHUNK = 128
_NCH = _PER_W // _CHUNK


def _body(h_idx, r_idx, t_idx, ent, rel, out_h, out_r, out_t,
          idx_v, rows_v, sem):
    wid = lax.axis_index("s") * _NC + lax.axis_index("c")
    base = wid * _PER_W

    def run_table(table, idx_hbm, out_hbm):
        pltpu.sync_copy(idx_hbm.at[pl.ds(base, _PER_W)], idx_v)

        def chunk(i, carry):
            off = pl.multiple_of(i * _CHUNK, _CHUNK)
            pltpu.async_copy(table.at[idx_v.at[pl.ds(off, _CHUNK)]],
                             rows_v, sem).wait()
            pltpu.sync_copy(rows_v, out_hbm.at[pl.ds(base + off, _CHUNK)])
            return carry

        lax.fori_loop(0, _NCH, chunk, 0)

    run_table(ent, h_idx, out_h)
    run_table(rel, r_idx, out_r)
    run_table(ent, t_idx, out_t)


@jax.jit
def _run(h_flat, r_flat, t_flat, ent, rel):
    mesh = plsc.VectorSubcoreMesh(
        core_axis_name="c", subcore_axis_name="s",
        num_cores=_NC, num_subcores=_NS)
    out = jax.ShapeDtypeStruct((_N, _D), jnp.float32)
    f = pl.kernel(
        _body,
        out_type=(out, out, out),
        mesh=mesh,
        scratch_types=[
            pltpu.VMEM((_PER_W,), jnp.int32),
            pltpu.VMEM((_CHUNK, _D), jnp.float32),
            pltpu.SemaphoreType.DMA,
        ],
        compiler_params=pltpu.CompilerParams(use_tc_tiling_on_sc=False),
    )
    return f(h_flat, r_flat, t_flat, ent, rel)


def kernel(h_id, r_id, t_id, ent_transfer, rel_transfer):
    h_flat = h_id.reshape(-1).astype(jnp.int32)
    r_flat = r_id.reshape(-1).astype(jnp.int32)
    t_flat = t_id.reshape(-1).astype(jnp.int32)
    oh, orr, ot = _run(h_flat, r_flat, t_flat,
                       ent_transfer, rel_transfer)
    shp = h_id.shape + (_D,)
    return (oh.reshape(shp), orr.reshape(shp), ot.reshape(shp))


# speedup vs baseline: 1.9458x; 1.0745x over previous
import functools

import jax
import jax.numpy as jnp
from jax import lax
from jax.experimental import pallas as pl
from jax.experimental.pallas import tpu as pltpu
from jax.experimental.pallas import tpu_sc as plsc

_B = 4096
_T = 50
_D = 64
_N = _B * _T
_NC = 2
_NS = 16
_NW = _NC * _NS
_PER_W = _N // _NW
_CHUNK = 800
_NCH = _PER_W // _CHUNK
_NBUF = 2


def _body(h_idx, r_idx, t_idx, ent, rel, out_h, out_r, out_t,
          idx_v, buf0, buf1, g0, g1, w0, w1):
    wid = lax.axis_index("s") * _NC + lax.axis_index("c")
    base = wid * _PER_W
    bufs = (buf0, buf1)
    gsems = (g0, g1)
    wsems = (w0, w1)

    pltpu.sync_copy(h_idx.at[pl.ds(base, _PER_W)], idx_v.at[0])
    pltpu.sync_copy(r_idx.at[pl.ds(base, _PER_W)], idx_v.at[1])
    pltpu.sync_copy(t_idx.at[pl.ds(base, _PER_W)], idx_v.at[2])

    jobs = []
    for t, (table, out_hbm) in enumerate(
            ((ent, out_h), (rel, out_r), (ent, out_t))):
        for c in range(_NCH):
            jobs.append((table, t, c, out_hbm))

    gdesc = [None] * _NBUF
    wdesc = [None] * _NBUF
    for j, (table, t, c, out_hbm) in enumerate(jobs):
        b = j % _NBUF
        if wdesc[b] is not None:
            wdesc[b].wait()
        gdesc[b] = pltpu.async_copy(
            table.at[idx_v.at[t, pl.ds(c * _CHUNK, _CHUNK)]],
            bufs[b], gsems[b])
        if j > 0:
            pj, pt, pc, pout = jobs[j - 1]
            pb = (j - 1) % _NBUF
            gdesc[pb].wait()
            wdesc[pb] = pltpu.async_copy(
                bufs[pb], pout.at[pl.ds(base + pc * _CHUNK, _CHUNK)],
                wsems[pb])
    lb = (len(jobs) - 1) % _NBUF
    gdesc[lb].wait()
    _, lt, lc, lout = jobs[-1]
    wdesc[lb] = pltpu.async_copy(
        bufs[lb], lout.at[pl.ds(base + lc * _CHUNK, _CHUNK)], wsems[lb])
    for d in wdesc:
        d.wait()


@jax.jit
def _run(h_flat, r_flat, t_flat, ent, rel):
    mesh = plsc.VectorSubcoreMesh(
        core_axis_name="c", subcore_axis_name="s",
        num_cores=_NC, num_subcores=_NS)
    out = jax.ShapeDtypeStruct((_N, _D), jnp.float32)
    f = pl.kernel(
        _body,
        out_type=(out, out, out),
        mesh=mesh,
        scratch_types=[
            pltpu.VMEM((3, _PER_W), jnp.int32),
            pltpu.VMEM((_CHUNK, _D), jnp.float32),
            pltpu.VMEM((_CHUNK, _D), jnp.float32),
            pltpu.SemaphoreType.DMA,
            pltpu.SemaphoreType.DMA,
            pltpu.SemaphoreType.DMA,
            pltpu.SemaphoreType.DMA,
        ],
        compiler_params=pltpu.CompilerParams(use_tc_tiling_on_sc=False),
    )
    return f(h_flat, r_flat, t_flat, ent, rel)


def kernel(h_id, r_id, t_id, ent_transfer, rel_transfer):
    h_flat = h_id.reshape(-1).astype(jnp.int32)
    r_flat = r_id.reshape(-1).astype(jnp.int32)
    t_flat = t_id.reshape(-1).astype(jnp.int32)
    oh, orr, ot = _run(h_flat, r_flat, t_flat,
                       ent_transfer, rel_transfer)
    shp = h_id.shape + (_D,)
    return (oh.reshape(shp), orr.reshape(shp), ot.reshape(shp))


# speedup vs baseline: 2.0394x; 1.0481x over previous
import functools

import jax
import jax.numpy as jnp
from jax import lax
from jax.experimental import pallas as pl
from jax.experimental.pallas import tpu as pltpu
from jax.experimental.pallas import tpu_sc as plsc

_B = 4096
_T = 50
_D = 64
_N = _B * _T
_NC = 2
_NS = 16
_NW = _NC * _NS
_PER_W = _N // _NW
_CHUNK = 800
_NCH = _PER_W // _CHUNK
_NBUF = 2


def _gather_body(idx_hbm, table, out_hbm, idx_v, buf0, buf1, g0, g1, w0, w1):
    wid = lax.axis_index("s") * _NC + lax.axis_index("c")
    base = wid * _PER_W
    bufs = (buf0, buf1)
    gsems = (g0, g1)
    wsems = (w0, w1)

    pltpu.sync_copy(idx_hbm.at[pl.ds(base, _PER_W)], idx_v)

    gdesc = [None] * _NBUF
    wdesc = [None] * _NBUF
    for c in range(_NCH):
        b = c % _NBUF
        if wdesc[b] is not None:
            wdesc[b].wait()
        gdesc[b] = pltpu.async_copy(
            table.at[idx_v.at[pl.ds(c * _CHUNK, _CHUNK)]], bufs[b], gsems[b])
        if c > 0:
            pb = (c - 1) % _NBUF
            gdesc[pb].wait()
            wdesc[pb] = pltpu.async_copy(
                bufs[pb], out_hbm.at[pl.ds(base + (c - 1) * _CHUNK, _CHUNK)],
                wsems[pb])
    lb = (_NCH - 1) % _NBUF
    gdesc[lb].wait()
    wdesc[lb] = pltpu.async_copy(
        bufs[lb], out_hbm.at[pl.ds(base + (_NCH - 1) * _CHUNK, _CHUNK)],
        wsems[lb])
    for d in wdesc:
        if d is not None:
            d.wait()


def _make_gather():
    mesh = plsc.VectorSubcoreMesh(
        core_axis_name="c", subcore_axis_name="s",
        num_cores=_NC, num_subcores=_NS)
    return pl.kernel(
        _gather_body,
        out_type=jax.ShapeDtypeStruct((_N, _D), jnp.float32),
        mesh=mesh,
        scratch_types=[
            pltpu.VMEM((_PER_W,), jnp.int32),
            pltpu.VMEM((_CHUNK, _D), jnp.float32),
            pltpu.VMEM((_CHUNK, _D), jnp.float32),
            pltpu.SemaphoreType.DMA,
            pltpu.SemaphoreType.DMA,
            pltpu.SemaphoreType.DMA,
            pltpu.SemaphoreType.DMA,
        ],
        compiler_params=pltpu.CompilerParams(use_tc_tiling_on_sc=False),
    )


@jax.jit
def _run(h_flat, r_flat, t_flat, ent, rel):
    ent2 = lax.optimization_barrier(ent.reshape(-1)).reshape(ent.shape)
    rel2 = lax.optimization_barrier(rel.reshape(-1)).reshape(rel.shape)
    gather = _make_gather()
    orr = gather(r_flat, rel2)
    oh = gather(h_flat, ent2)
    ot = gather(t_flat, ent2)
    return oh, orr, ot


def kernel(h_id, r_id, t_id, ent_transfer, rel_transfer):
    h_flat = h_id.reshape(-1).astype(jnp.int32)
    r_flat = r_id.reshape(-1).astype(jnp.int32)
    t_flat = t_id.reshape(-1).astype(jnp.int32)
    oh, orr, ot = _run(h_flat, r_flat, t_flat,
                       ent_transfer, rel_transfer)
    shp = h_id.shape + (_D,)
    return (oh.reshape(shp), orr.reshape(shp), ot.reshape(shp))
